# Initial kernel scaffold; baseline (speedup 1.0000x reference)
import jax
import jax.numpy as jnp
from jax.experimental import pallas as pl


def kernel(x, temp, AD_indices, AD_values, W1, bias1, W2, bias2, a1, a2):
    raise NotImplementedError("write your pallas kernel here")


# speedup vs baseline: 2.7141x; 2.7141x over previous
import functools

import jax
import jax.numpy as jnp
import numpy as np
from jax import lax
from jax.experimental import pallas as pl
from jax.experimental.pallas import tpu as pltpu
from jax.experimental.pallas import tpu_sc as plsc

N = 10000
E = 160000
D = 256
H = 256
HALF = H // 2

NT = 16
NP = 10240
ROWS_PER_TILE = NP // NT
EDGES_PER_TILE = E // NT
K = 80
NCH = EDGES_PER_TILE // K
BN = 1000

_GDN = lax.GatherDimensionNumbers(
    offset_dims=(), collapsed_slice_dims=(0,), start_index_map=(0,))


def _lane_splat(vec, e16):
    idx = lax.full((16, 1), e16, jnp.int32)
    return lax.gather(vec, idx, _GDN, slice_sizes=(1,),
                      mode=lax.GatherScatterMode.PROMISE_IN_BOUNDS)


def _lin1_body(x_ref, w_ref, b_ref, a_ref, h0_ref, h1_ref):
    h = jnp.dot(x_ref[...], w_ref[...], preferred_element_type=jnp.float32)
    h = h + b_ref[...]
    h = jnp.where(h > 0, h, a_ref[...] * h)
    h0_ref[...] = h[:, :HALF]
    h1_ref[...] = h[:, HALF:]


def _linear1(x, w1t, b1, a1):
    return pl.pallas_call(
        _lin1_body,
        grid=(N // BN,),
        in_specs=[
            pl.BlockSpec((BN, D), lambda i: (i, 0)),
            pl.BlockSpec((D, H), lambda i: (0, 0)),
            pl.BlockSpec((1, H), lambda i: (0, 0)),
            pl.BlockSpec((1, H), lambda i: (0, 0)),
        ],
        out_specs=[
            pl.BlockSpec((BN, HALF), lambda i: (i, 0)),
            pl.BlockSpec((BN, HALF), lambda i: (i, 0)),
        ],
        out_shape=[
            jax.ShapeDtypeStruct((N, HALF), jnp.float32),
            jax.ShapeDtypeStruct((N, HALF), jnp.float32),
        ],
    )(x, w1t, b1, a1)


def _lin2_body(g0_ref, g1_ref, w0_ref, w1_ref, b_ref, a_ref, o_ref):
    t = jnp.dot(g0_ref[...], w0_ref[...], preferred_element_type=jnp.float32)
    t = t + jnp.dot(g1_ref[...], w1_ref[...], preferred_element_type=jnp.float32)
    t = t + b_ref[...]
    o_ref[...] = jnp.where(t > 0, t, a_ref[...] * t)


def _linear2(g0, g1, w2a, w2b, b2, a2):
    return pl.pallas_call(
        _lin2_body,
        grid=(N // BN,),
        in_specs=[
            pl.BlockSpec((BN, HALF), lambda i: (i, 0)),
            pl.BlockSpec((BN, HALF), lambda i: (i, 0)),
            pl.BlockSpec((HALF, H), lambda i: (0, 0)),
            pl.BlockSpec((HALF, H), lambda i: (0, 0)),
            pl.BlockSpec((1, H), lambda i: (0, 0)),
            pl.BlockSpec((1, H), lambda i: (0, 0)),
        ],
        out_specs=pl.BlockSpec((BN, H), lambda i: (i, 0)),
        out_shape=jax.ShapeDtypeStruct((N, H), jnp.float32),
    )(g0, g1, w2a, w2b, b2, a2)


def _sc_agg_body(h0, h1, src3, dst3, vals3, agg0, agg1,
                 sidx, didx, valv, rows, shared, sem):
    c = lax.axis_index("c")
    s = lax.axis_index("s")
    rbase = s * ROWS_PER_TILE

    def work(h_ref, agg_ref):
        zvec = lax.full((16,), 0.0, jnp.float32)
        for r in range(K):
            for t in range(HALF // 16):
                rows[r, pl.ds(t * 16, 16)] = zvec
        for b in range(ROWS_PER_TILE // K):
            pltpu.sync_copy(rows, shared.at[pl.ds(rbase + b * K, K)])
        plsc.subcore_barrier()

        def chunk(j, carry):
            pltpu.sync_copy(src3.at[s, j], sidx)
            pltpu.sync_copy(dst3.at[s, j], didx)
            pltpu.sync_copy(vals3.at[s, j], valv)
            pltpu.async_copy(h_ref.at[sidx], rows, sem).wait()

            def group(g, cc):
                vgrp = valv[pl.ds(g * 16, 16)]
                for e16 in range(16):
                    vv = _lane_splat(vgrp, e16)
                    e = g * 16 + e16
                    for t in range(HALF // 16):
                        sl = pl.ds(t * 16, 16)
                        rows[e, sl] = rows[e, sl] * vv
                return cc

            lax.fori_loop(0, K // 16, group, 0)
            pltpu.sync_copy(rows, shared.at[didx], add=True)
            return carry

        lax.fori_loop(0, NCH, chunk, 0)
        plsc.subcore_barrier()
        pltpu.sync_copy(shared.at[pl.ds(rbase, ROWS_PER_TILE)],
                        agg_ref.at[pl.ds(rbase, ROWS_PER_TILE)])

    @pl.when(c == 0)
    def _():
        work(h0, agg0)

    @pl.when(c == 1)
    def _():
        work(h1, agg1)


_sc_agg = functools.partial(
    pl.kernel,
    out_type=[
        jax.ShapeDtypeStruct((NP, HALF), jnp.float32),
        jax.ShapeDtypeStruct((NP, HALF), jnp.float32),
    ],
    mesh=plsc.VectorSubcoreMesh(core_axis_name="c", subcore_axis_name="s"),
    scratch_types=[
        pltpu.VMEM((K,), jnp.int32),
        pltpu.VMEM((K,), jnp.int32),
        pltpu.VMEM((K,), jnp.float32),
        pltpu.VMEM((K, HALF), jnp.float32),
        pltpu.VMEM_SHARED((NP, HALF), jnp.float32),
        pltpu.SemaphoreType.DMA,
    ],
)(_sc_agg_body)


def kernel(x, temp, AD_indices, AD_values, W1, bias1, W2, bias2, a1, a2):
    idx = AD_indices.astype(jnp.int32)
    dst3 = idx[0].reshape(NT, NCH, K)
    src3 = idx[1].reshape(NT, NCH, K)
    vals3 = AD_values.reshape(NT, NCH, K)

    h0, h1 = _linear1(x, W1.T, bias1.reshape(1, H), a1.reshape(1, H))

    agg0, agg1 = _sc_agg(h0, h1, src3, dst3, vals3)
    agg0, agg1 = agg0[:N], agg1[:N]

    return _linear2(agg0, agg1, W2[:, :HALF].T, W2[:, HALF:].T,
                    bias2.reshape(1, H), a2.reshape(1, H))
